# baseline (speedup 1.0000x reference)
import functools

import jax
import jax.numpy as jnp
from jax import lax
from jax.experimental import pallas as pl
from jax.experimental.pallas import tpu as pltpu
from jax.experimental.pallas import tpu_sc as plsc

VOCAB = 1000
EMBED = 64
B = 4096
L = 50
VPAD = 1024

NC = 2
NS = 16
NW = NC * NS
BW = B // NW
LANES = 16


def _body(idx_hbm, wt_hbm, out_hbm, table_v, idx_v, buf0, buf1, sem0, sem1):
    wid = lax.axis_index("s") * NC + lax.axis_index("c")
    b0 = wid * BW
    pltpu.sync_copy(wt_hbm, table_v)
    pltpu.sync_copy(idx_hbm.at[:, pl.ds(b0, BW)], idx_v)

    bufs = (buf0, buf1)
    sems = (sem0, sem1)

    def owait(k):
        pltpu.make_async_copy(
            bufs[k], out_hbm.at[pl.ds(0, EMBED), pl.ds(b0, BW)], sems[k]
        ).wait()

    def ostart(l, k):
        pltpu.async_copy(
            bufs[k], out_hbm.at[pl.ds(l * EMBED, EMBED), pl.ds(b0, BW)],
            sems[k],
        )

    def fill(l, buf):
        idxs = tuple(idx_v[l, pl.ds(g * LANES, LANES)]
                     for g in range(BW // LANES))

        @plsc.parallel_loop(0, EMBED, unroll=8, carry=idxs)
        def e_body(e, idxs):
            off = e * VPAD
            for g in range(BW // LANES):
                v = plsc.load_gather(table_v, [idxs[g] + off])
                buf[e, pl.ds(g * LANES, LANES)] = v
            return idxs

    def pair(r, carry):
        for k in range(2):
            l = 2 * r + k

            @pl.when(r > 0)
            def _():
                owait(k)

            fill(l, bufs[k])
            ostart(l, k)
        return carry

    lax.fori_loop(0, L // 2, pair, 0)
    owait(0)
    owait(1)


@functools.partial(
    pl.kernel,
    out_type=jax.ShapeDtypeStruct((L * EMBED, B), jnp.float32),
    mesh=plsc.VectorSubcoreMesh(core_axis_name="c", subcore_axis_name="s"),
    scratch_types=[
        pltpu.VMEM((EMBED * VPAD,), jnp.float32),
        pltpu.VMEM((L, BW), jnp.int32),
        pltpu.VMEM((EMBED, BW), jnp.float32),
        pltpu.VMEM((EMBED, BW), jnp.float32),
        pltpu.SemaphoreType.DMA,
        pltpu.SemaphoreType.DMA,
    ],
    compiler_params=pltpu.CompilerParams(use_tc_tiling_on_sc=True,
                                         needs_layout_passes=False),
)
def _gather_kernel(idx_hbm, wt_hbm, out_hbm, table_v, idx_v, buf0, buf1,
                   sem0, sem1):
    _body(idx_hbm, wt_hbm, out_hbm, table_v, idx_v, buf0, buf1, sem0, sem1)


def kernel(indices, W):
    idx_t = indices.T
    wt = jnp.pad(W.T, ((0, 0), (0, VPAD - VOCAB)))
    wt_flat = wt.reshape(EMBED * VPAD)
    out2 = _gather_kernel(idx_t, wt_flat)
    return out2.reshape(L, EMBED, B).transpose(2, 0, 1)


# speedup vs baseline: 1.0187x; 1.0187x over previous
import functools

import jax
import jax.numpy as jnp
from jax import lax
from jax.experimental import pallas as pl
from jax.experimental.pallas import tpu as pltpu
from jax.experimental.pallas import tpu_sc as plsc

VOCAB = 1000
EMBED = 64
B = 4096
L = 50
VPAD = 1024

NC = 2
NS = 16
NW = NC * NS
BW = B // NW
LANES = 16


def _body(idx_hbm, wt_hbm, out_hbm, table_v, idx_v, buf0, buf1, sem0, sem1):
    wid = lax.axis_index("s") * NC + lax.axis_index("c")
    b0 = wid * BW
    pltpu.sync_copy(wt_hbm, table_v)
    pltpu.sync_copy(idx_hbm.at[:, pl.ds(b0, BW)], idx_v)

    bufs = (buf0, buf1)
    sems = (sem0, sem1)

    def owait(k):
        pltpu.make_async_copy(
            bufs[k], out_hbm.at[pl.ds(0, EMBED), pl.ds(b0, BW)], sems[k]
        ).wait()

    def ostart(l, k):
        pltpu.async_copy(
            bufs[k], out_hbm.at[pl.ds(l * EMBED, EMBED), pl.ds(b0, BW)],
            sems[k],
        )

    def fill(l, buf):
        idxs = tuple(idx_v[l, pl.ds(g * LANES, LANES)]
                     for g in range(BW // LANES))

        @plsc.parallel_loop(0, EMBED, unroll=4, carry=idxs)
        def e_body(e, idxs):
            off = e * VPAD
            for g in range(BW // LANES):
                v = plsc.load_gather(table_v, [idxs[g] + off])
                buf[e, pl.ds(g * LANES, LANES)] = v
            return idxs

    def pair(r, carry):
        for k in range(2):
            l = 2 * r + k

            @pl.when(r > 0)
            def _():
                owait(k)

            fill(l, bufs[k])
            ostart(l, k)
        return carry

    lax.fori_loop(0, L // 2, pair, 0)
    owait(0)
    owait(1)


@functools.partial(
    pl.kernel,
    out_type=jax.ShapeDtypeStruct((L * EMBED, B), jnp.float32),
    mesh=plsc.VectorSubcoreMesh(core_axis_name="c", subcore_axis_name="s"),
    scratch_types=[
        pltpu.VMEM((EMBED * VPAD,), jnp.float32),
        pltpu.VMEM((L, BW), jnp.int32),
        pltpu.VMEM((EMBED, BW), jnp.float32),
        pltpu.VMEM((EMBED, BW), jnp.float32),
        pltpu.SemaphoreType.DMA,
        pltpu.SemaphoreType.DMA,
    ],
    compiler_params=pltpu.CompilerParams(use_tc_tiling_on_sc=True,
                                         needs_layout_passes=False),
)
def _gather_kernel(idx_hbm, wt_hbm, out_hbm, table_v, idx_v, buf0, buf1,
                   sem0, sem1):
    _body(idx_hbm, wt_hbm, out_hbm, table_v, idx_v, buf0, buf1, sem0, sem1)


def kernel(indices, W):
    idx_t = indices.T
    wt = jnp.pad(W.T, ((0, 0), (0, VPAD - VOCAB)))
    wt_flat = wt.reshape(EMBED * VPAD)
    out2 = _gather_kernel(idx_t, wt_flat)
    return out2.reshape(L, EMBED, B).transpose(2, 0, 1)


# speedup vs baseline: 1.0229x; 1.0042x over previous
import functools

import jax
import jax.numpy as jnp
from jax import lax
from jax.experimental import pallas as pl
from jax.experimental.pallas import tpu as pltpu
from jax.experimental.pallas import tpu_sc as plsc

VOCAB = 1000
EMBED = 64
B = 4096
L = 50
VPAD = 1024

NC = 2
NS = 16
NW = NC * NS
BW = B // NW
LANES = 16


def _body(idx_hbm, wt_hbm, out_hbm, table_v, idx_v, buf0, buf1, sem0, sem1):
    wid = lax.axis_index("s") * NC + lax.axis_index("c")
    b0 = wid * BW
    pltpu.async_copy(wt_hbm, table_v, sem0)
    pltpu.async_copy(idx_hbm.at[:, pl.ds(b0, BW)], idx_v, sem1)
    pltpu.make_async_copy(wt_hbm, table_v, sem0).wait()
    pltpu.make_async_copy(idx_hbm.at[:, pl.ds(b0, BW)], idx_v, sem1).wait()

    bufs = (buf0, buf1)
    sems = (sem0, sem1)

    def owait(k):
        pltpu.make_async_copy(
            bufs[k], out_hbm.at[pl.ds(0, EMBED), pl.ds(b0, BW)], sems[k]
        ).wait()

    def ostart(l, k):
        pltpu.async_copy(
            bufs[k], out_hbm.at[pl.ds(l * EMBED, EMBED), pl.ds(b0, BW)],
            sems[k],
        )

    def fill(l, buf):
        idxs = tuple(idx_v[l, pl.ds(g * LANES, LANES)]
                     for g in range(BW // LANES))

        @plsc.parallel_loop(0, EMBED, unroll=4, carry=idxs)
        def e_body(e, idxs):
            off = e * VPAD
            for g in range(BW // LANES):
                v = plsc.load_gather(table_v, [idxs[g] + off])
                buf[e, pl.ds(g * LANES, LANES)] = v
            return idxs

    def pair(r, carry):
        for k in range(2):
            l = 2 * r + k

            @pl.when(r > 0)
            def _():
                owait(k)

            fill(l, bufs[k])
            ostart(l, k)
        return carry

    lax.fori_loop(0, L // 2, pair, 0)
    owait(0)
    owait(1)


@functools.partial(
    pl.kernel,
    out_type=jax.ShapeDtypeStruct((L * EMBED, B), jnp.float32),
    mesh=plsc.VectorSubcoreMesh(core_axis_name="c", subcore_axis_name="s"),
    scratch_types=[
        pltpu.VMEM((EMBED * VPAD,), jnp.float32),
        pltpu.VMEM((L, BW), jnp.int32),
        pltpu.VMEM((EMBED, BW), jnp.float32),
        pltpu.VMEM((EMBED, BW), jnp.float32),
        pltpu.SemaphoreType.DMA,
        pltpu.SemaphoreType.DMA,
    ],
    compiler_params=pltpu.CompilerParams(use_tc_tiling_on_sc=True,
                                         needs_layout_passes=False),
)
def _gather_kernel(idx_hbm, wt_hbm, out_hbm, table_v, idx_v, buf0, buf1,
                   sem0, sem1):
    _body(idx_hbm, wt_hbm, out_hbm, table_v, idx_v, buf0, buf1, sem0, sem1)


def kernel(indices, W):
    idx_t = indices.T
    wt = jnp.pad(W.T, ((0, 0), (0, VPAD - VOCAB)))
    wt_flat = wt.reshape(EMBED * VPAD)
    out2 = _gather_kernel(idx_t, wt_flat)
    return out2.reshape(L, EMBED, B).transpose(2, 0, 1)


# speedup vs baseline: 1.0287x; 1.0056x over previous
import functools

import jax
import jax.numpy as jnp
from jax import lax
from jax.experimental import pallas as pl
from jax.experimental.pallas import tpu as pltpu
from jax.experimental.pallas import tpu_sc as plsc

VOCAB = 1000
EMBED = 64
B = 4096
L = 50
VPAD = 1024

NC = 2
NS = 16
NW = NC * NS
BW = B // NW
LANES = 16


def _body(idx_hbm, wt_hbm, out_hbm, table_v, idx_v, buf0, buf1, sem0, sem1):
    wid = lax.axis_index("s") * NC + lax.axis_index("c")
    b0 = wid * BW
    pltpu.async_copy(wt_hbm, table_v, sem0)
    pltpu.async_copy(idx_hbm.at[:, pl.ds(b0, BW)], idx_v, sem1)
    pltpu.make_async_copy(wt_hbm, table_v, sem0).wait()
    pltpu.make_async_copy(idx_hbm.at[:, pl.ds(b0, BW)], idx_v, sem1).wait()

    bufs = (buf0, buf1)
    sems = (sem0, sem1)

    def owait(k):
        pltpu.make_async_copy(
            bufs[k], out_hbm.at[pl.ds(0, 2 * EMBED), pl.ds(b0, BW)], sems[k]
        ).wait()

    def ostart(p, k):
        pltpu.async_copy(
            bufs[k],
            out_hbm.at[pl.ds(p * 2 * EMBED, 2 * EMBED), pl.ds(b0, BW)],
            sems[k],
        )

    def fill(l, buf, half):
        idxs = tuple(idx_v[l, pl.ds(g * LANES, LANES)]
                     for g in range(BW // LANES))
        row0 = half * EMBED

        @plsc.parallel_loop(0, EMBED, unroll=4, carry=idxs)
        def e_body(e, idxs):
            off = e * VPAD
            for g in range(BW // LANES):
                v = plsc.load_gather(table_v, [idxs[g] + off])
                buf[row0 + e, pl.ds(g * LANES, LANES)] = v
            return idxs

    def duo(r, carry):
        for k in range(2):
            p = 2 * r + k

            @pl.when(r > 0)
            def _():
                owait(k)

            fill(2 * p, bufs[k], 0)
            fill(2 * p + 1, bufs[k], 1)
            ostart(p, k)
        return carry

    lax.fori_loop(0, L // 4, duo, 0)
    owait(0)
    fill(48, bufs[0], 0)
    fill(49, bufs[0], 1)
    ostart(24, 0)
    owait(0)
    owait(1)


@functools.partial(
    pl.kernel,
    out_type=jax.ShapeDtypeStruct((L * EMBED, B), jnp.float32),
    mesh=plsc.VectorSubcoreMesh(core_axis_name="c", subcore_axis_name="s"),
    scratch_types=[
        pltpu.VMEM((EMBED * VPAD,), jnp.float32),
        pltpu.VMEM((L, BW), jnp.int32),
        pltpu.VMEM((2 * EMBED, BW), jnp.float32),
        pltpu.VMEM((2 * EMBED, BW), jnp.float32),
        pltpu.SemaphoreType.DMA,
        pltpu.SemaphoreType.DMA,
    ],
    compiler_params=pltpu.CompilerParams(use_tc_tiling_on_sc=True,
                                         needs_layout_passes=False),
)
def _gather_kernel(idx_hbm, wt_hbm, out_hbm, table_v, idx_v, buf0, buf1,
                   sem0, sem1):
    _body(idx_hbm, wt_hbm, out_hbm, table_v, idx_v, buf0, buf1, sem0, sem1)


def kernel(indices, W):
    idx_t = indices.T
    wt = jnp.pad(W.T, ((0, 0), (0, VPAD - VOCAB)))
    wt_flat = wt.reshape(EMBED * VPAD)
    out2 = _gather_kernel(idx_t, wt_flat)
    return out2.reshape(L, EMBED, B).transpose(2, 0, 1)
